# baseline (speedup 1.0000x reference)
import functools

import jax
import jax.numpy as jnp
from jax import lax
from jax.experimental import pallas as pl
from jax.experimental.pallas import tpu as pltpu
from jax.experimental.pallas import tpu_sc as plsc

_CHUNKS = 4


def _make_sc_copy(rows, cols, dtype):
    info = plsc.get_sparse_core_info()
    nc, ns = info.num_cores, info.num_subcores
    nw = nc * ns
    r_per_w = rows // nw
    r_per_c = r_per_w // _CHUNKS
    mesh = plsc.VectorSubcoreMesh(core_axis_name="c", subcore_axis_name="s")

    @functools.partial(
        pl.kernel,
        mesh=mesh,
        out_type=jax.ShapeDtypeStruct((rows, cols), dtype),
        scratch_types=[
            pltpu.VMEM((_CHUNKS, r_per_c, cols), dtype),
            pltpu.SemaphoreType.DMA((_CHUNKS,)),
            pltpu.SemaphoreType.DMA((_CHUNKS,)),
        ],
    )
    def k(x_hbm, out_hbm, bufs, sem_in, sem_out):
        wid = lax.axis_index("s") * nc + lax.axis_index("c")
        base = wid * r_per_w
        in_copies = []
        for g in range(_CHUNKS):
            c = pltpu.make_async_copy(
                x_hbm.at[pl.ds(base + g * r_per_c, r_per_c)],
                bufs.at[g],
                sem_in.at[g],
            )
            c.start()
            in_copies.append(c)
        out_copies = []
        for g in range(_CHUNKS):
            in_copies[g].wait()
            c = pltpu.make_async_copy(
                bufs.at[g],
                out_hbm.at[pl.ds(base + g * r_per_c, r_per_c)],
                sem_out.at[g],
            )
            c.start()
            out_copies.append(c)
        for c in out_copies:
            c.wait()

    return k


def kernel(x):
    rows, cols = x.shape
    return _make_sc_copy(rows, cols, x.dtype)(x)


# speedup vs baseline: 4.2334x; 4.2334x over previous
import jax
from jax.experimental import pallas as pl
from jax.experimental.pallas import tpu as pltpu

_BLOCK_ROWS = 8192


def _copy_kernel(x_ref, o_ref):
    o_ref[...] = x_ref[...]


def kernel(x):
    rows, cols = x.shape
    grid = rows // _BLOCK_ROWS
    return pl.pallas_call(
        _copy_kernel,
        out_shape=jax.ShapeDtypeStruct(x.shape, x.dtype),
        grid=(grid,),
        in_specs=[pl.BlockSpec((_BLOCK_ROWS, cols), lambda i: (i, 0))],
        out_specs=pl.BlockSpec((_BLOCK_ROWS, cols), lambda i: (i, 0)),
        compiler_params=pltpu.CompilerParams(
            dimension_semantics=("arbitrary",),
        ),
    )(x)
